# Initial kernel scaffold; baseline (speedup 1.0000x reference)
import jax
import jax.numpy as jnp
from jax.experimental import pallas as pl


def kernel(cls_heads, reg_heads, center_heads, controllers_heads, mask_out, batch_positions):
    raise NotImplementedError("write your pallas kernel here")


# speedup vs baseline: 11.7807x; 11.7807x over previous
import functools

import jax
import jax.numpy as jnp
from jax import lax
from jax.experimental import pallas as pl
from jax.experimental.pallas import tpu as pltpu

MASK_STRIDE = 8
NUM_MASKS = 8
TOPN = 1000
MIN_SCORE = 0.1
NMS_THR = 0.6
MASK_THR = 0.5
MAX_OBJ = 100
NEG_INF = float("-inf")


def _decode_body(cls_ref, cen_ref, reg_ref, pos_ref, ms_ref, cls_out_ref, box_ref):
    c = cls_ref[0]
    smax = jnp.max(c, axis=1, keepdims=True)
    iot = lax.broadcasted_iota(jnp.int32, c.shape, 1)
    amax = jnp.min(jnp.where(c == smax, iot, c.shape[1]), axis=1, keepdims=True)
    cen = cen_ref[0]
    s = jnp.sqrt(smax * cen)
    ms_ref[0] = jnp.where(s > MIN_SCORE, s, NEG_INF)
    cls_out_ref[0] = amax.astype(jnp.float32)
    reg = reg_ref[0]
    pos = pos_ref[0]
    box_ref[0] = jnp.concatenate(
        [pos - reg[:, 0:2], pos + reg[:, 2:4]], axis=1)


def _nms_body(ts_ref, tsc_ref, ti_ref, box_ref, cls_ref, eye_ref,
              ks_ref, kc_ref, kb_ref, fn_ref,
              cx1, cy1, cx2, cy2, s_scr, kcs):
    P = 1024
    zc = jnp.zeros((P, 1), jnp.float32)
    cx1[...] = zc
    cy1[...] = zc
    cx2[...] = zc
    cy2[...] = zc
    kcs[...] = zc

    def gather(i, _):
        idx = ti_ref[0, 0, i]
        row = box_ref[0, pl.ds(idx, 1), :]
        cx1[pl.ds(i, 1), :] = row[:, 0:1]
        cy1[pl.ds(i, 1), :] = row[:, 1:2]
        cx2[pl.ds(i, 1), :] = row[:, 2:3]
        cy2[pl.ds(i, 1), :] = row[:, 3:4]

        @pl.when(i < MAX_OBJ)
        def _():
            kcs[pl.ds(i, 1), :] = cls_ref[0, pl.ds(idx, 1), :]
        return 0

    lax.fori_loop(0, TOPN, gather, 0)

    ax1, ay1, ax2, ay2 = cx1[...], cy1[...], cx2[...], cy2[...]
    c4 = jnp.concatenate([ax1, ay1, ax2, ay2], axis=1)
    r4 = lax.dot_general(c4, eye_ref[...], (((0,), (0,)), ((), ())),
                         precision=lax.Precision.HIGHEST)
    bx1, by1, bx2, by2 = r4[0:1, :], r4[1:2, :], r4[2:3, :], r4[3:4, :]
    area_c = jnp.clip(ax2 - ax1, 0.0, None) * jnp.clip(ay2 - ay1, 0.0, None)
    area_r = jnp.clip(bx2 - bx1, 0.0, None) * jnp.clip(by2 - by1, 0.0, None)

    CH = 64
    for cidx in range(P // CH):
        lo, hi = cidx * CH, (cidx + 1) * CH
        xx1 = jnp.maximum(ax1[lo:hi, :], bx1)
        yy1 = jnp.maximum(ay1[lo:hi, :], by1)
        xx2 = jnp.minimum(ax2[lo:hi, :], bx2)
        yy2 = jnp.minimum(ay2[lo:hi, :], by2)
        inter = jnp.clip(xx2 - xx1, 0.0, None) * jnp.clip(yy2 - yy1, 0.0, None)
        union = area_c[lo:hi, :] + area_r - inter
        iou = inter / jnp.maximum(union, 1e-6)
        ri = lax.broadcasted_iota(jnp.int32, (CH, P), 0) + lo
        ci = lax.broadcasted_iota(jnp.int32, (CH, P), 1)
        s_scr[pl.ds(lo, CH), :] = jnp.where((iou > NMS_THR) & (ci > ri), 1.0, 0.0)

    ts = ts_ref[0]
    lidx = lax.broadcasted_iota(jnp.int32, (1, P), 1)
    keep0 = jnp.where(ts > MIN_SCORE, 1.0, 0.0)

    def nms_step(i, kvec):
        srow = s_scr[pl.ds(i, 1), :]
        ki = jnp.max(jnp.where(lidx == i, kvec, 0.0), axis=1, keepdims=True)
        return kvec * (1.0 - ki * srow)

    keep = lax.fori_loop(0, TOPN, nms_step, keep0)
    nkeep = jnp.sum(keep, axis=1, keepdims=True)
    fn = jnp.minimum(jnp.float32(MAX_OBJ), nkeep).astype(jnp.int32)
    fn_ref[pl.program_id(0), 0] = fn[0, 0]

    fnf = fn.astype(jnp.int32)
    r100 = lax.broadcasted_iota(jnp.int32, (MAX_OBJ, 1), 0)
    slot_b = r100 < fnf
    ks_ref[0] = jnp.where(slot_b, tsc_ref[0][0:MAX_OBJ, :], -1.0)
    kc_ref[0] = jnp.where(slot_b, kcs[0:MAX_OBJ, :], -1.0)
    slot_c = jnp.where(slot_b, 1.0, 0.0)
    tb = jnp.concatenate([cx1[0:MAX_OBJ, :], cy1[0:MAX_OBJ, :],
                          cx2[0:MAX_OBJ, :], cy2[0:MAX_OBJ, :]], axis=1)
    kb_ref[0] = tb * slot_c


def _masks_body(ctl_ref, fa_ref, u_ref, ut_ref, ti_ref, fn_ref,
                out_ref, p_scr):
    n = pl.program_id(1)
    idx = ti_ref[0, 0, n]
    row = ctl_ref[0, pl.ds(idx, 1), :]
    nm = NUM_MASKS

    w1_rows = []
    for m in range(nm):
        w1_rows.append(jnp.concatenate(
            [row[:, m * 10:(m + 1) * 10], row[:, 80 + m:81 + m],
             jnp.zeros((1, 21), jnp.float32)], axis=1))
    w1 = jnp.concatenate(w1_rows, axis=0)

    fa = fa_ref[0]
    e10 = jnp.where(lax.broadcasted_iota(jnp.int32, (32, 1), 0) == 10,
                    1.0, 0.0)
    b1c = lax.dot_general(w1, e10, (((1,), (0,)), ((), ())),
                          precision=lax.Precision.HIGHEST)
    x1 = jnp.maximum(lax.dot_general(
        w1, fa, (((1,), (0,)), ((), ()))) + b1c, 0.0)

    w2_rows = []
    for m in range(nm):
        w2_rows.append(jnp.concatenate(
            [row[:, 88 + m * 8:96 + m * 8], row[:, 152 + m:153 + m],
             jnp.zeros((1, 7), jnp.float32)], axis=1))
    w2 = jnp.concatenate(w2_rows, axis=0)
    e8 = jnp.where(lax.broadcasted_iota(jnp.int32, (16, 1), 0) == 8,
                   1.0, 0.0)
    b2c = lax.dot_general(w2, e8, (((1,), (0,)), ((), ())),
                          precision=lax.Precision.HIGHEST)

    x1a = jnp.concatenate(
        [x1, jnp.zeros((8, 4096), jnp.float32)], axis=0)
    x2 = jnp.maximum(lax.dot_general(
        w2, x1a, (((1,), (0,)), ((), ()))) + b2c, 0.0)

    w3 = jnp.concatenate([row[:, 160:168], row[:, 168:169]], axis=1)
    x2a = jnp.concatenate(
        [x2, jnp.zeros((1, 4096), jnp.float32)], axis=0)
    logits = lax.dot_general(
        w3, x2a, (((1,), (0,)), ((), ()))) + row[:, 168:169]
    probs = jax.nn.sigmoid(logits)

    for r in range(64):
        p_scr[pl.ds(r, 1), :] = probs[:, r * 64:(r + 1) * 64]

    u = u_ref[...]
    ut = ut_ref[...]
    t1 = lax.dot_general(u, p_scr[...], (((1,), (0,)), ((), ())),
                         precision=lax.Precision.HIGHEST)
    valid = n < fn_ref[pl.program_id(0), 0]
    for c in range(8):
        chunk = lax.dot_general(t1[c * 64:(c + 1) * 64, :], ut,
                                (((1,), (0,)), ((), ())),
                                precision=lax.Precision.HIGHEST)
        out_ref[0, 0, pl.ds(c * 64, 64), :] = jnp.logical_and(
            chunk > MASK_THR, valid)


def _upsample_operator(src, dst):
    y = (jnp.arange(dst, dtype=jnp.float32) + 0.5) / (dst / src) - 0.5
    yc = jnp.clip(y, 0.0, src - 1.0)
    y0 = jnp.minimum(jnp.floor(yc), src - 2.0)
    f = yc - y0
    cols = jnp.arange(src, dtype=jnp.float32)[None, :]
    u = (jnp.where(cols == y0[:, None], 1.0 - f[:, None], 0.0)
         + jnp.where(cols == y0[:, None] + 1.0, f[:, None], 0.0))
    return u.astype(jnp.float32)


def kernel(cls_heads, reg_heads, center_heads, controllers_heads, mask_out,
           batch_positions):
    B, N, C = cls_heads.shape
    H = W = 64
    HW = H * W

    ms, classes, boxes = pl.pallas_call(
        _decode_body,
        grid=(B,),
        in_specs=[
            pl.BlockSpec((1, N, C), lambda b: (b, 0, 0)),
            pl.BlockSpec((1, N, 1), lambda b: (b, 0, 0)),
            pl.BlockSpec((1, N, 4), lambda b: (b, 0, 0)),
            pl.BlockSpec((1, N, 2), lambda b: (b, 0, 0)),
        ],
        out_specs=[
            pl.BlockSpec((1, N, 1), lambda b: (b, 0, 0)),
            pl.BlockSpec((1, N, 1), lambda b: (b, 0, 0)),
            pl.BlockSpec((1, N, 4), lambda b: (b, 0, 0)),
        ],
        out_shape=[
            jax.ShapeDtypeStruct((B, N, 1), jnp.float32),
            jax.ShapeDtypeStruct((B, N, 1), jnp.float32),
            jax.ShapeDtypeStruct((B, N, 4), jnp.float32),
        ],
    )(cls_heads, center_heads, reg_heads, batch_positions)

    top_s, top_i = lax.top_k(ms.reshape(B, N), TOPN)
    ts_pad = jnp.pad(top_s, ((0, 0), (0, 1024 - TOPN)),
                     constant_values=-1.0).reshape(B, 1, 1024)
    ts_col = ts_pad.reshape(B, 1024, 1)
    ti = top_i.astype(jnp.int32).reshape(B, 1, TOPN)
    eye = jnp.eye(1024, dtype=jnp.float32)

    ks, kc, kb, fn = pl.pallas_call(
        _nms_body,
        grid=(B,),
        in_specs=[
            pl.BlockSpec((1, 1, 1024), lambda b: (b, 0, 0)),
            pl.BlockSpec((1, 1024, 1), lambda b: (b, 0, 0)),
            pl.BlockSpec((1, 1, TOPN), lambda b: (b, 0, 0),
                         memory_space=pltpu.SMEM),
            pl.BlockSpec((1, N, 4), lambda b: (b, 0, 0)),
            pl.BlockSpec((1, N, 1), lambda b: (b, 0, 0)),
            pl.BlockSpec((1024, 1024), lambda b: (0, 0)),
        ],
        out_specs=[
            pl.BlockSpec((1, MAX_OBJ, 1), lambda b: (b, 0, 0)),
            pl.BlockSpec((1, MAX_OBJ, 1), lambda b: (b, 0, 0)),
            pl.BlockSpec((1, MAX_OBJ, 4), lambda b: (b, 0, 0)),
            pl.BlockSpec((4, 1), lambda b: (0, 0), memory_space=pltpu.SMEM),
        ],
        out_shape=[
            jax.ShapeDtypeStruct((B, MAX_OBJ, 1), jnp.float32),
            jax.ShapeDtypeStruct((B, MAX_OBJ, 1), jnp.float32),
            jax.ShapeDtypeStruct((B, MAX_OBJ, 4), jnp.float32),
            jax.ShapeDtypeStruct((B, 1), jnp.int32),
        ],
        scratch_shapes=[pltpu.VMEM((1024, 1), jnp.float32)] * 4
        + [pltpu.VMEM((1024, 1024), jnp.float32),
           pltpu.VMEM((1024, 1), jnp.float32)],
    )(ts_pad, ts_col, ti, boxes, classes, eye)

    mo_t = jnp.transpose(mask_out, (0, 3, 1, 2)).reshape(B, NUM_MASKS, HW)
    cx = (jnp.arange(W, dtype=jnp.float32) / (W - 1)) * 2.0 - 1.0
    cy = (jnp.arange(H, dtype=jnp.float32) / (H - 1)) * 2.0 - 1.0
    cx_row = jnp.tile(cx, H).reshape(1, 1, HW)
    cy_row = jnp.repeat(cy, W).reshape(1, 1, HW)
    zeros_rows = jnp.zeros((1, 32 - NUM_MASKS - 2, HW), jnp.float32)
    fa = jnp.concatenate(
        [mo_t,
         jnp.broadcast_to(cx_row, (B, 1, HW)),
         jnp.broadcast_to(cy_row, (B, 1, HW)),
         jnp.broadcast_to(zeros_rows, (B, 32 - NUM_MASKS - 2, HW))], axis=1)

    u = _upsample_operator(H, H * MASK_STRIDE)
    ut = u.T

    masks = pl.pallas_call(
        _masks_body,
        grid=(B, MAX_OBJ),
        in_specs=[
            pl.BlockSpec((1, N, 169), lambda b, n: (b, 0, 0)),
            pl.BlockSpec((1, 32, HW), lambda b, n: (b, 0, 0)),
            pl.BlockSpec((512, 64), lambda b, n: (0, 0)),
            pl.BlockSpec((64, 512), lambda b, n: (0, 0)),
            pl.BlockSpec((1, 1, TOPN), lambda b, n: (b, 0, 0),
                         memory_space=pltpu.SMEM),
            pl.BlockSpec((4, 1), lambda b, n: (0, 0), memory_space=pltpu.SMEM),
        ],
        out_specs=pl.BlockSpec((1, 1, 512, 512), lambda b, n: (b, n, 0, 0)),
        out_shape=jax.ShapeDtypeStruct((B, MAX_OBJ, 512, 512), jnp.bool_),
        scratch_shapes=[pltpu.VMEM((64, 64), jnp.float32)],
    )(controllers_heads, fa, u, ut, ti, fn)

    return (ks.reshape(B, MAX_OBJ), kc.reshape(B, MAX_OBJ), masks,
            kb.reshape(B, MAX_OBJ, 4))


# speedup vs baseline: 14.6468x; 1.2433x over previous
import functools

import jax
import jax.numpy as jnp
from jax import lax
from jax.experimental import pallas as pl
from jax.experimental.pallas import tpu as pltpu

MASK_STRIDE = 8
NUM_MASKS = 8
TOPN = 1000
MIN_SCORE = 0.1
NMS_THR = 0.6
MASK_THR = 0.5
MAX_OBJ = 100
NEG_INF = float("-inf")


def _decode_body(cls_ref, cen_ref, reg_ref, pos_ref, ms_ref, cls_out_ref, box_ref):
    c = cls_ref[0]
    smax = jnp.max(c, axis=1, keepdims=True)
    iot = lax.broadcasted_iota(jnp.int32, c.shape, 1)
    amax = jnp.min(jnp.where(c == smax, iot, c.shape[1]), axis=1, keepdims=True)
    cen = cen_ref[0]
    s = jnp.sqrt(smax * cen)
    ms_ref[0] = jnp.where(s > MIN_SCORE, s, NEG_INF)
    cls_out_ref[0] = amax.astype(jnp.float32)
    reg = reg_ref[0]
    pos = pos_ref[0]
    box_ref[0] = jnp.concatenate(
        [pos - reg[:, 0:2], pos + reg[:, 2:4]], axis=1)


def _nms_body(ts_ref, tsc_ref, ti_ref, box_ref, cls_ref, eye_ref,
              ks_ref, kc_ref, kb_ref, fn_ref,
              tbox, s_scr, kcs):
    P = 1024
    tbox[...] = jnp.zeros((P, 4), jnp.float32)
    kcs[...] = jnp.zeros((P, 1), jnp.float32)

    def gather(i, _):
        idx = ti_ref[0, 0, i]
        tbox[pl.ds(i, 1), :] = box_ref[0, pl.ds(idx, 1), :]

        @pl.when(i < MAX_OBJ)
        def _():
            kcs[pl.ds(i, 1), :] = cls_ref[0, pl.ds(idx, 1), :]
        return 0

    lax.fori_loop(0, TOPN, gather, 0)

    c4 = tbox[...]
    ax1, ay1 = c4[:, 0:1], c4[:, 1:2]
    ax2, ay2 = c4[:, 2:3], c4[:, 3:4]
    r4 = lax.dot_general(c4, eye_ref[...], (((0,), (0,)), ((), ())),
                         precision=lax.Precision.HIGHEST)
    bx1, by1, bx2, by2 = r4[0:1, :], r4[1:2, :], r4[2:3, :], r4[3:4, :]
    area_c = jnp.clip(ax2 - ax1, 0.0, None) * jnp.clip(ay2 - ay1, 0.0, None)
    area_r = jnp.clip(bx2 - bx1, 0.0, None) * jnp.clip(by2 - by1, 0.0, None)

    CH = 64
    for cidx in range(P // CH):
        lo, hi = cidx * CH, (cidx + 1) * CH
        xx1 = jnp.maximum(ax1[lo:hi, :], bx1)
        yy1 = jnp.maximum(ay1[lo:hi, :], by1)
        xx2 = jnp.minimum(ax2[lo:hi, :], bx2)
        yy2 = jnp.minimum(ay2[lo:hi, :], by2)
        inter = jnp.clip(xx2 - xx1, 0.0, None) * jnp.clip(yy2 - yy1, 0.0, None)
        union = area_c[lo:hi, :] + area_r - inter
        iou = inter / jnp.maximum(union, 1e-6)
        ri = lax.broadcasted_iota(jnp.int32, (CH, P), 0) + lo
        ci = lax.broadcasted_iota(jnp.int32, (CH, P), 1)
        s_scr[pl.ds(lo, CH), :] = jnp.where((iou > NMS_THR) & (ci > ri), 1.0, 0.0)

    ts = ts_ref[0]
    lidx = lax.broadcasted_iota(jnp.int32, (1, P), 1)
    keep0 = jnp.where(ts > MIN_SCORE, 1.0, 0.0)

    def nms_step(i, kvec):
        srow = s_scr[pl.ds(i, 1), :]
        ki = jnp.max(jnp.where(lidx == i, kvec, 0.0), axis=1, keepdims=True)
        return kvec * (1.0 - ki * srow)

    keep = lax.fori_loop(0, TOPN, nms_step, keep0)
    nkeep = jnp.sum(keep, axis=1, keepdims=True)
    fn = jnp.minimum(jnp.float32(MAX_OBJ), nkeep).astype(jnp.int32)
    fn_ref[pl.program_id(0), 0] = fn[0, 0]

    fnf = fn.astype(jnp.int32)
    r100 = lax.broadcasted_iota(jnp.int32, (MAX_OBJ, 1), 0)
    slot_b = r100 < fnf
    ks_ref[0] = jnp.where(slot_b, tsc_ref[0][0:MAX_OBJ, :], -1.0)
    kc_ref[0] = jnp.where(slot_b, kcs[0:MAX_OBJ, :], -1.0)
    slot_c = jnp.where(slot_b, 1.0, 0.0)
    kb_ref[0] = tbox[0:MAX_OBJ, :] * slot_c


def _masks_body(ctl_ref, fa_ref, u_ref, ut_ref, ti_ref, fn_ref,
                out_ref, p_scr, *, G):
    nm = NUM_MASKS
    fa = fa_ref[0]
    fn = fn_ref[pl.program_id(0), 0]
    n0 = pl.program_id(1) * G

    rows = []
    w1_all = []
    for g in range(G):
        idx = ti_ref[0, 0, n0 + g]
        row = ctl_ref[0, pl.ds(idx, 1), :]
        rows.append(row)
        for m in range(nm):
            w1_all.append(jnp.concatenate(
                [row[:, m * 10:(m + 1) * 10], row[:, 80 + m:81 + m],
                 jnp.zeros((1, 21), jnp.float32)], axis=1))
    w1 = jnp.concatenate(w1_all, axis=0)
    e10 = jnp.where(lax.broadcasted_iota(jnp.int32, (32, 1), 0) == 10,
                    1.0, 0.0)
    b1c = lax.dot_general(w1, e10, (((1,), (0,)), ((), ())),
                          precision=lax.Precision.HIGHEST)
    x1 = jnp.maximum(lax.dot_general(
        w1, fa, (((1,), (0,)), ((), ()))) + b1c, 0.0)

    u = u_ref[...]
    ut = ut_ref[...]
    e8 = jnp.where(lax.broadcasted_iota(jnp.int32, (16, 1), 0) == 8,
                   1.0, 0.0)
    for g in range(G):
        row = rows[g]
        w2_rows = []
        for m in range(nm):
            w2_rows.append(jnp.concatenate(
                [row[:, 88 + m * 8:96 + m * 8], row[:, 152 + m:153 + m],
                 jnp.zeros((1, 7), jnp.float32)], axis=1))
        w2 = jnp.concatenate(w2_rows, axis=0)
        b2c = lax.dot_general(w2, e8, (((1,), (0,)), ((), ())),
                              precision=lax.Precision.HIGHEST)
        x1a = jnp.concatenate(
            [x1[g * nm:(g + 1) * nm, :],
             jnp.zeros((8, 4096), jnp.float32)], axis=0)
        x2 = jnp.maximum(lax.dot_general(
            w2, x1a, (((1,), (0,)), ((), ()))) + b2c, 0.0)

        w3 = jnp.concatenate([row[:, 160:168], row[:, 168:169]], axis=1)
        x2a = jnp.concatenate(
            [x2, jnp.zeros((1, 4096), jnp.float32)], axis=0)
        logits = lax.dot_general(
            w3, x2a, (((1,), (0,)), ((), ()))) + row[:, 168:169]
        probs = jax.nn.sigmoid(logits)

        for r in range(64):
            p_scr[pl.ds(r, 1), :] = probs[:, r * 64:(r + 1) * 64]

        t1 = lax.dot_general(u, p_scr[...], (((1,), (0,)), ((), ())),
                             precision=lax.Precision.HIGHEST)
        valid = (n0 + g) < fn
        for c in range(8):
            chunk = lax.dot_general(t1[c * 64:(c + 1) * 64, :], ut,
                                    (((1,), (0,)), ((), ())),
                                    precision=lax.Precision.HIGHEST)
            out_ref[0, g, pl.ds(c * 64, 64), :] = jnp.logical_and(
                chunk > MASK_THR, valid)


def _upsample_operator(src, dst):
    y = (jnp.arange(dst, dtype=jnp.float32) + 0.5) / (dst / src) - 0.5
    yc = jnp.clip(y, 0.0, src - 1.0)
    y0 = jnp.minimum(jnp.floor(yc), src - 2.0)
    f = yc - y0
    cols = jnp.arange(src, dtype=jnp.float32)[None, :]
    u = (jnp.where(cols == y0[:, None], 1.0 - f[:, None], 0.0)
         + jnp.where(cols == y0[:, None] + 1.0, f[:, None], 0.0))
    return u.astype(jnp.float32)


def kernel(cls_heads, reg_heads, center_heads, controllers_heads, mask_out,
           batch_positions):
    B, N, C = cls_heads.shape
    H = W = 64
    HW = H * W

    ms, classes, boxes = pl.pallas_call(
        _decode_body,
        grid=(B,),
        in_specs=[
            pl.BlockSpec((1, N, C), lambda b: (b, 0, 0)),
            pl.BlockSpec((1, N, 1), lambda b: (b, 0, 0)),
            pl.BlockSpec((1, N, 4), lambda b: (b, 0, 0)),
            pl.BlockSpec((1, N, 2), lambda b: (b, 0, 0)),
        ],
        out_specs=[
            pl.BlockSpec((1, N, 1), lambda b: (b, 0, 0)),
            pl.BlockSpec((1, N, 1), lambda b: (b, 0, 0)),
            pl.BlockSpec((1, N, 4), lambda b: (b, 0, 0)),
        ],
        out_shape=[
            jax.ShapeDtypeStruct((B, N, 1), jnp.float32),
            jax.ShapeDtypeStruct((B, N, 1), jnp.float32),
            jax.ShapeDtypeStruct((B, N, 4), jnp.float32),
        ],
    )(cls_heads, center_heads, reg_heads, batch_positions)

    top_s, top_i = lax.top_k(ms.reshape(B, N), TOPN)
    ts_pad = jnp.pad(top_s, ((0, 0), (0, 1024 - TOPN)),
                     constant_values=-1.0).reshape(B, 1, 1024)
    ts_col = ts_pad.reshape(B, 1024, 1)
    ti = top_i.astype(jnp.int32).reshape(B, 1, TOPN)
    eye = jnp.eye(1024, dtype=jnp.float32)

    ks, kc, kb, fn = pl.pallas_call(
        _nms_body,
        grid=(B,),
        in_specs=[
            pl.BlockSpec((1, 1, 1024), lambda b: (b, 0, 0)),
            pl.BlockSpec((1, 1024, 1), lambda b: (b, 0, 0)),
            pl.BlockSpec((1, 1, TOPN), lambda b: (b, 0, 0),
                         memory_space=pltpu.SMEM),
            pl.BlockSpec((1, N, 4), lambda b: (b, 0, 0)),
            pl.BlockSpec((1, N, 1), lambda b: (b, 0, 0)),
            pl.BlockSpec((1024, 1024), lambda b: (0, 0)),
        ],
        out_specs=[
            pl.BlockSpec((1, MAX_OBJ, 1), lambda b: (b, 0, 0)),
            pl.BlockSpec((1, MAX_OBJ, 1), lambda b: (b, 0, 0)),
            pl.BlockSpec((1, MAX_OBJ, 4), lambda b: (b, 0, 0)),
            pl.BlockSpec((4, 1), lambda b: (0, 0), memory_space=pltpu.SMEM),
        ],
        out_shape=[
            jax.ShapeDtypeStruct((B, MAX_OBJ, 1), jnp.float32),
            jax.ShapeDtypeStruct((B, MAX_OBJ, 1), jnp.float32),
            jax.ShapeDtypeStruct((B, MAX_OBJ, 4), jnp.float32),
            jax.ShapeDtypeStruct((B, 1), jnp.int32),
        ],
        scratch_shapes=[pltpu.VMEM((1024, 4), jnp.float32),
                        pltpu.VMEM((1024, 1024), jnp.float32),
                        pltpu.VMEM((1024, 1), jnp.float32)],
    )(ts_pad, ts_col, ti, boxes, classes, eye)

    mo_t = jnp.transpose(mask_out, (0, 3, 1, 2)).reshape(B, NUM_MASKS, HW)
    cx = (jnp.arange(W, dtype=jnp.float32) / (W - 1)) * 2.0 - 1.0
    cy = (jnp.arange(H, dtype=jnp.float32) / (H - 1)) * 2.0 - 1.0
    cx_row = jnp.tile(cx, H).reshape(1, 1, HW)
    cy_row = jnp.repeat(cy, W).reshape(1, 1, HW)
    zeros_rows = jnp.zeros((1, 32 - NUM_MASKS - 2, HW), jnp.float32)
    fa = jnp.concatenate(
        [mo_t,
         jnp.broadcast_to(cx_row, (B, 1, HW)),
         jnp.broadcast_to(cy_row, (B, 1, HW)),
         jnp.broadcast_to(zeros_rows, (B, 32 - NUM_MASKS - 2, HW))], axis=1)

    u = _upsample_operator(H, H * MASK_STRIDE)
    ut = u.T

    G = 4
    masks = pl.pallas_call(
        functools.partial(_masks_body, G=G),
        grid=(B, MAX_OBJ // G),
        in_specs=[
            pl.BlockSpec((1, N, 169), lambda b, n: (b, 0, 0)),
            pl.BlockSpec((1, 32, HW), lambda b, n: (b, 0, 0)),
            pl.BlockSpec((512, 64), lambda b, n: (0, 0)),
            pl.BlockSpec((64, 512), lambda b, n: (0, 0)),
            pl.BlockSpec((1, 1, TOPN), lambda b, n: (b, 0, 0),
                         memory_space=pltpu.SMEM),
            pl.BlockSpec((4, 1), lambda b, n: (0, 0), memory_space=pltpu.SMEM),
        ],
        out_specs=pl.BlockSpec((1, G, 512, 512), lambda b, n: (b, n, 0, 0)),
        out_shape=jax.ShapeDtypeStruct((B, MAX_OBJ, 512, 512), jnp.bool_),
        scratch_shapes=[pltpu.VMEM((64, 64), jnp.float32)],
    )(controllers_heads, fa, u, ut, ti, fn)

    return (ks.reshape(B, MAX_OBJ), kc.reshape(B, MAX_OBJ), masks,
            kb.reshape(B, MAX_OBJ, 4))


# speedup vs baseline: 14.8738x; 1.0155x over previous
import functools

import jax
import jax.numpy as jnp
from jax import lax
from jax.experimental import pallas as pl
from jax.experimental.pallas import tpu as pltpu

MASK_STRIDE = 8
NUM_MASKS = 8
TOPN = 1000
MIN_SCORE = 0.1
NMS_THR = 0.6
MASK_THR = 0.5
MAX_OBJ = 100
NEG_INF = float("-inf")


def _decode_body(cls_ref, cen_ref, reg_ref, pos_ref, ms_ref, cls_out_ref, box_ref):
    c = cls_ref[0]
    smax = jnp.max(c, axis=1, keepdims=True)
    iot = lax.broadcasted_iota(jnp.int32, c.shape, 1)
    amax = jnp.min(jnp.where(c == smax, iot, c.shape[1]), axis=1, keepdims=True)
    cen = cen_ref[0]
    s = jnp.sqrt(smax * cen)
    ms_ref[0] = jnp.where(s > MIN_SCORE, s, NEG_INF)
    cls_out_ref[0] = amax.astype(jnp.float32)
    reg = reg_ref[0]
    pos = pos_ref[0]
    box_ref[0] = jnp.concatenate(
        [pos - reg[:, 0:2], pos + reg[:, 2:4]], axis=1)


def _nms_body(ts_ref, tsc_ref, ti_ref, box_ref, cls_ref, eye_ref,
              ks_ref, kc_ref, kb_ref, fn_ref,
              tbox, s_scr, kcs):
    P = 1024
    tbox[...] = jnp.zeros((P, 4), jnp.float32)
    kcs[...] = jnp.zeros((P, 1), jnp.float32)

    def gather(i, _):
        idx = ti_ref[0, 0, i]
        tbox[pl.ds(i, 1), :] = box_ref[0, pl.ds(idx, 1), :]

        @pl.when(i < MAX_OBJ)
        def _():
            kcs[pl.ds(i, 1), :] = cls_ref[0, pl.ds(idx, 1), :]
        return 0

    lax.fori_loop(0, TOPN, gather, 0)

    c4 = tbox[...]
    ax1, ay1 = c4[:, 0:1], c4[:, 1:2]
    ax2, ay2 = c4[:, 2:3], c4[:, 3:4]
    r4 = lax.dot_general(c4, eye_ref[...], (((0,), (0,)), ((), ())),
                         precision=lax.Precision.HIGHEST)
    bx1, by1, bx2, by2 = r4[0:1, :], r4[1:2, :], r4[2:3, :], r4[3:4, :]
    area_c = jnp.clip(ax2 - ax1, 0.0, None) * jnp.clip(ay2 - ay1, 0.0, None)
    area_r = jnp.clip(bx2 - bx1, 0.0, None) * jnp.clip(by2 - by1, 0.0, None)

    CH = 64
    for cidx in range(P // CH):
        lo, hi = cidx * CH, (cidx + 1) * CH
        xx1 = jnp.maximum(ax1[lo:hi, :], bx1)
        yy1 = jnp.maximum(ay1[lo:hi, :], by1)
        xx2 = jnp.minimum(ax2[lo:hi, :], bx2)
        yy2 = jnp.minimum(ay2[lo:hi, :], by2)
        inter = jnp.clip(xx2 - xx1, 0.0, None) * jnp.clip(yy2 - yy1, 0.0, None)
        union = area_c[lo:hi, :] + area_r - inter
        iou = inter / jnp.maximum(union, 1e-6)
        ri = lax.broadcasted_iota(jnp.int32, (CH, P), 0) + lo
        ci = lax.broadcasted_iota(jnp.int32, (CH, P), 1)
        s_scr[pl.ds(lo, CH), :] = jnp.where((iou > NMS_THR) & (ci > ri), 1.0, 0.0)

    ts = ts_ref[0]
    lidx = lax.broadcasted_iota(jnp.int32, (1, P), 1)
    keep0 = jnp.where(ts > MIN_SCORE, 1.0, 0.0)

    def nms_step(i, kvec):
        srow = s_scr[pl.ds(i, 1), :]
        ki = jnp.max(jnp.where(lidx == i, kvec, 0.0), axis=1, keepdims=True)
        return kvec * (1.0 - ki * srow)

    keep = lax.fori_loop(0, TOPN, nms_step, keep0)
    nkeep = jnp.sum(keep, axis=1, keepdims=True)
    fn = jnp.minimum(jnp.float32(MAX_OBJ), nkeep).astype(jnp.int32)
    fn_ref[pl.program_id(0), 0] = fn[0, 0]

    fnf = fn.astype(jnp.int32)
    r100 = lax.broadcasted_iota(jnp.int32, (MAX_OBJ, 1), 0)
    slot_b = r100 < fnf
    ks_ref[0] = jnp.where(slot_b, tsc_ref[0][0:MAX_OBJ, :], -1.0)
    kc_ref[0] = jnp.where(slot_b, kcs[0:MAX_OBJ, :], -1.0)
    slot_c = jnp.where(slot_b, 1.0, 0.0)
    kb_ref[0] = tbox[0:MAX_OBJ, :] * slot_c


def _masks_body(ctl_ref, fa_ref, u_ref, ut_ref, ti_ref, fn_ref,
                out_ref, p_scr, *, G):
    nm = NUM_MASKS
    fa = fa_ref[0]
    fn = fn_ref[pl.program_id(0), 0]
    n0 = pl.program_id(1) * G

    rows = []
    w1_all = []
    for g in range(G):
        idx = ti_ref[0, 0, n0 + g]
        row = ctl_ref[0, pl.ds(idx, 1), :]
        rows.append(row)
        for m in range(nm):
            w1_all.append(jnp.concatenate(
                [row[:, m * 10:(m + 1) * 10], row[:, 80 + m:81 + m],
                 jnp.zeros((1, 21), jnp.float32)], axis=1))
    w1 = jnp.concatenate(w1_all, axis=0)
    e10 = jnp.where(lax.broadcasted_iota(jnp.int32, (32, 1), 0) == 10,
                    1.0, 0.0)
    b1c = lax.dot_general(w1, e10, (((1,), (0,)), ((), ())),
                          precision=lax.Precision.HIGHEST)
    x1 = jnp.maximum(lax.dot_general(
        w1, fa, (((1,), (0,)), ((), ()))) + b1c, 0.0)

    u = u_ref[...]
    ut = ut_ref[...]
    e8 = jnp.where(lax.broadcasted_iota(jnp.int32, (16, 1), 0) == 8,
                   1.0, 0.0)
    for g in range(G):
        row = rows[g]
        w2_rows = []
        for m in range(nm):
            w2_rows.append(jnp.concatenate(
                [row[:, 88 + m * 8:96 + m * 8], row[:, 152 + m:153 + m],
                 jnp.zeros((1, 7), jnp.float32)], axis=1))
        w2 = jnp.concatenate(w2_rows, axis=0)
        b2c = lax.dot_general(w2, e8, (((1,), (0,)), ((), ())),
                              precision=lax.Precision.HIGHEST)
        x1a = jnp.concatenate(
            [x1[g * nm:(g + 1) * nm, :],
             jnp.zeros((8, 4096), jnp.float32)], axis=0)
        x2 = jnp.maximum(lax.dot_general(
            w2, x1a, (((1,), (0,)), ((), ()))) + b2c, 0.0)

        w3 = jnp.concatenate([row[:, 160:168], row[:, 168:169]], axis=1)
        x2a = jnp.concatenate(
            [x2, jnp.zeros((1, 4096), jnp.float32)], axis=0)
        logits = lax.dot_general(
            w3, x2a, (((1,), (0,)), ((), ()))) + row[:, 168:169]
        probs = jax.nn.sigmoid(logits)

        for r in range(64):
            p_scr[pl.ds(r, 1), :] = probs[:, r * 64:(r + 1) * 64]

        t1 = lax.dot_general(u, p_scr[...], (((1,), (0,)), ((), ())),
                             precision=lax.Precision.HIGHEST)
        valid = (n0 + g) < fn
        for c in range(8):
            chunk = lax.dot_general(t1[c * 64:(c + 1) * 64, :], ut,
                                    (((1,), (0,)), ((), ())),
                                    precision=lax.Precision.HIGHEST)
            out_ref[0, g, pl.ds(c * 64, 64), :] = jnp.logical_and(
                chunk > MASK_THR, valid)


def _upsample_operator(src, dst):
    y = (jnp.arange(dst, dtype=jnp.float32) + 0.5) / (dst / src) - 0.5
    yc = jnp.clip(y, 0.0, src - 1.0)
    y0 = jnp.minimum(jnp.floor(yc), src - 2.0)
    f = yc - y0
    cols = jnp.arange(src, dtype=jnp.float32)[None, :]
    u = (jnp.where(cols == y0[:, None], 1.0 - f[:, None], 0.0)
         + jnp.where(cols == y0[:, None] + 1.0, f[:, None], 0.0))
    return u.astype(jnp.float32)


def kernel(cls_heads, reg_heads, center_heads, controllers_heads, mask_out,
           batch_positions):
    B, N, C = cls_heads.shape
    H = W = 64
    HW = H * W

    ms, classes, boxes = pl.pallas_call(
        _decode_body,
        grid=(B,),
        in_specs=[
            pl.BlockSpec((1, N, C), lambda b: (b, 0, 0)),
            pl.BlockSpec((1, N, 1), lambda b: (b, 0, 0)),
            pl.BlockSpec((1, N, 4), lambda b: (b, 0, 0)),
            pl.BlockSpec((1, N, 2), lambda b: (b, 0, 0)),
        ],
        out_specs=[
            pl.BlockSpec((1, N, 1), lambda b: (b, 0, 0)),
            pl.BlockSpec((1, N, 1), lambda b: (b, 0, 0)),
            pl.BlockSpec((1, N, 4), lambda b: (b, 0, 0)),
        ],
        out_shape=[
            jax.ShapeDtypeStruct((B, N, 1), jnp.float32),
            jax.ShapeDtypeStruct((B, N, 1), jnp.float32),
            jax.ShapeDtypeStruct((B, N, 4), jnp.float32),
        ],
    )(cls_heads, center_heads, reg_heads, batch_positions)

    top_s, top_i = lax.top_k(ms.reshape(B, N), TOPN)
    ts_pad = jnp.pad(top_s, ((0, 0), (0, 1024 - TOPN)),
                     constant_values=-1.0).reshape(B, 1, 1024)
    ts_col = ts_pad.reshape(B, 1024, 1)
    ti = top_i.astype(jnp.int32).reshape(B, 1, TOPN)
    eye = jnp.eye(1024, dtype=jnp.float32)

    ks, kc, kb, fn = pl.pallas_call(
        _nms_body,
        grid=(B,),
        in_specs=[
            pl.BlockSpec((1, 1, 1024), lambda b: (b, 0, 0)),
            pl.BlockSpec((1, 1024, 1), lambda b: (b, 0, 0)),
            pl.BlockSpec((1, 1, TOPN), lambda b: (b, 0, 0),
                         memory_space=pltpu.SMEM),
            pl.BlockSpec((1, N, 4), lambda b: (b, 0, 0)),
            pl.BlockSpec((1, N, 1), lambda b: (b, 0, 0)),
            pl.BlockSpec((1024, 1024), lambda b: (0, 0)),
        ],
        out_specs=[
            pl.BlockSpec((1, MAX_OBJ, 1), lambda b: (b, 0, 0)),
            pl.BlockSpec((1, MAX_OBJ, 1), lambda b: (b, 0, 0)),
            pl.BlockSpec((1, MAX_OBJ, 4), lambda b: (b, 0, 0)),
            pl.BlockSpec((4, 1), lambda b: (0, 0), memory_space=pltpu.SMEM),
        ],
        out_shape=[
            jax.ShapeDtypeStruct((B, MAX_OBJ, 1), jnp.float32),
            jax.ShapeDtypeStruct((B, MAX_OBJ, 1), jnp.float32),
            jax.ShapeDtypeStruct((B, MAX_OBJ, 4), jnp.float32),
            jax.ShapeDtypeStruct((B, 1), jnp.int32),
        ],
        scratch_shapes=[pltpu.VMEM((1024, 4), jnp.float32),
                        pltpu.VMEM((1024, 1024), jnp.float32),
                        pltpu.VMEM((1024, 1), jnp.float32)],
    )(ts_pad, ts_col, ti, boxes, classes, eye)

    mo_t = jnp.transpose(mask_out, (0, 3, 1, 2)).reshape(B, NUM_MASKS, HW)
    cx = (jnp.arange(W, dtype=jnp.float32) / (W - 1)) * 2.0 - 1.0
    cy = (jnp.arange(H, dtype=jnp.float32) / (H - 1)) * 2.0 - 1.0
    cx_row = jnp.tile(cx, H).reshape(1, 1, HW)
    cy_row = jnp.repeat(cy, W).reshape(1, 1, HW)
    zeros_rows = jnp.zeros((1, 32 - NUM_MASKS - 2, HW), jnp.float32)
    fa = jnp.concatenate(
        [mo_t,
         jnp.broadcast_to(cx_row, (B, 1, HW)),
         jnp.broadcast_to(cy_row, (B, 1, HW)),
         jnp.broadcast_to(zeros_rows, (B, 32 - NUM_MASKS - 2, HW))], axis=1)

    u = _upsample_operator(H, H * MASK_STRIDE)
    ut = u.T

    G = 10
    masks = pl.pallas_call(
        functools.partial(_masks_body, G=G),
        grid=(B, MAX_OBJ // G),
        in_specs=[
            pl.BlockSpec((1, N, 169), lambda b, n: (b, 0, 0)),
            pl.BlockSpec((1, 32, HW), lambda b, n: (b, 0, 0)),
            pl.BlockSpec((512, 64), lambda b, n: (0, 0)),
            pl.BlockSpec((64, 512), lambda b, n: (0, 0)),
            pl.BlockSpec((1, 1, TOPN), lambda b, n: (b, 0, 0),
                         memory_space=pltpu.SMEM),
            pl.BlockSpec((4, 1), lambda b, n: (0, 0), memory_space=pltpu.SMEM),
        ],
        out_specs=pl.BlockSpec((1, G, 512, 512), lambda b, n: (b, n, 0, 0)),
        out_shape=jax.ShapeDtypeStruct((B, MAX_OBJ, 512, 512), jnp.bool_),
        scratch_shapes=[pltpu.VMEM((64, 64), jnp.float32)],
    )(controllers_heads, fa, u, ut, ti, fn)

    return (ks.reshape(B, MAX_OBJ), kc.reshape(B, MAX_OBJ), masks,
            kb.reshape(B, MAX_OBJ, 4))
